# Initial kernel scaffold; baseline (speedup 1.0000x reference)
import jax
import jax.numpy as jnp
from jax.experimental import pallas as pl


def kernel(pred, y_hat):
    raise NotImplementedError("write your pallas kernel here")


# speedup vs baseline: 7.4203x; 7.4203x over previous
import functools

import numpy as np
import jax
import jax.numpy as jnp
from jax import lax
from jax.experimental import pallas as pl
from jax.experimental.pallas import tpu as pltpu
from jax.experimental.pallas import tpu_sc as plsc

_PRIORS = (np.array([[1.3221, 1.73145], [3.19275, 4.00944], [5.05587, 8.09892],
                     [9.47112, 4.84053], [11.2364, 10.0071]],
                    dtype=np.float32) / 13.0)
_IMG = 416.0
_B, _C, _H, _W = 64, 125, 52, 52
_DX = _IMG / float(_C)
_NW = 32
_BPW = _B // _NW
_COLS = (0, 16, 32, 36)
_NCH = 5
_NBUF = 3


def _trunc(x):
    return x.astype(jnp.int32).astype(jnp.float32)


def _row_cls(buf, m_v, r, planes):
    ms = [m_v[r, pl.ds(c, 16)] for c in _COLS]
    for k in planes:
        xs = [buf[k, r, pl.ds(c, 16)] for c in _COLS]
        for c, x, m in zip(_COLS, xs, ms):
            buf[k, r, pl.ds(c, 16)] = x * m


def _row_special(buf, m_v, bx_v, by_v, r, wa, ha):
    ms = [m_v[r, pl.ds(c, 16)] for c in _COLS]
    xs = [buf[0, r, pl.ds(c, 16)] for c in _COLS]
    for c, x, m in zip(_COLS, xs, ms):
        buf[0, r, pl.ds(c, 16)] = x * m
    xs = [buf[1, r, pl.ds(c, 16)] for c in _COLS]
    bxs = [bx_v[r, pl.ds(c, 16)] for c in _COLS]
    for c, x, bx, m in zip(_COLS, xs, bxs, ms):
        buf[1, r, pl.ds(c, 16)] = (bx + _trunc(_DX * x)) * m
    xs = [buf[2, r, pl.ds(c, 16)] for c in _COLS]
    bys = [by_v[r, pl.ds(c, 16)] for c in _COLS]
    for c, x, by, m in zip(_COLS, xs, bys, ms):
        buf[2, r, pl.ds(c, 16)] = (by + _trunc(_DX * x)) * m
    xs = [buf[3, r, pl.ds(c, 16)] for c in _COLS]
    for c, x, m in zip(_COLS, xs, ms):
        buf[3, r, pl.ds(c, 16)] = _trunc((wa * x) * _IMG) * m
    xs = [buf[4, r, pl.ds(c, 16)] for c in _COLS]
    for c, x, m in zip(_COLS, xs, ms):
        buf[4, r, pl.ds(c, 16)] = _trunc((ha * x) * _IMG) * m


def _make_sc_call():
    mesh = plsc.VectorSubcoreMesh(core_axis_name="c", subcore_axis_name="s")

    @functools.partial(
        pl.kernel, mesh=mesh,
        out_type=jax.ShapeDtypeStruct((_B, _C, _H, _W), jnp.float32),
        scratch_types=[
            pltpu.VMEM((_H, _W), jnp.float32),
            pltpu.VMEM((_H, _W), jnp.float32),
            pltpu.VMEM((_H, _W), jnp.float32),
            [pltpu.VMEM((_NCH, _H, _W), jnp.float32)] * _NBUF,
            [pltpu.SemaphoreType.DMA] * _NBUF,
            [pltpu.SemaphoreType.DMA] * _NBUF,
        ],
    )
    def _k(pred_hbm, y0_hbm, bx_hbm, by_hbm, out_hbm,
           bx_v, by_v, m_v, bufs, isems, osems):
        wid = lax.axis_index("s") * 2 + lax.axis_index("c")
        pltpu.sync_copy(bx_hbm, bx_v)
        pltpu.sync_copy(by_hbm, by_v)

        nchunks = _C // _NCH

        def batch_body(bi, carry):
            b = wid * _BPW + bi

            for j in range(_NBUF):
                pltpu.async_copy(pred_hbm.at[b, pl.ds(j * _NCH, _NCH)],
                                 bufs[j], isems[j])
            pltpu.sync_copy(y0_hbm.at[b], m_v)

            def mbody(r, c2):
                ys = [m_v[r, pl.ds(c, 16)] for c in _COLS]
                for c, y in zip(_COLS, ys):
                    m_v[r, pl.ds(c, 16)] = 5.0 * y + 0.5 * (1.0 - y)
                return c2
            lax.fori_loop(0, _H, mbody, 0)

            for j in range(nchunks):
                s = j % _NBUF
                c0 = j * _NCH
                a, part = divmod(j, _NCH)
                buf = bufs[s]
                pltpu.make_async_copy(pred_hbm.at[b, pl.ds(c0, _NCH)],
                                      buf, isems[s]).wait()
                if part == 0:
                    wa = float(_PRIORS[a, 0])
                    ha = float(_PRIORS[a, 1])

                    def sbody(r, c2, buf=buf, wa=wa, ha=ha):
                        _row_special(buf, m_v, bx_v, by_v, r, wa, ha)
                        return c2
                    lax.fori_loop(0, _H, sbody, 0)
                else:
                    def cbody(r, c2, buf=buf):
                        _row_cls(buf, m_v, r, range(_NCH))
                        return c2
                    lax.fori_loop(0, _H, cbody, 0)
                pltpu.async_copy(buf, out_hbm.at[b, pl.ds(c0, _NCH)], osems[s])
                nj = j + _NBUF
                if nj < nchunks:
                    pltpu.make_async_copy(buf, out_hbm.at[b, pl.ds(c0, _NCH)],
                                          osems[s]).wait()
                    pltpu.async_copy(pred_hbm.at[b, pl.ds(nj * _NCH, _NCH)],
                                     buf, isems[s])

            for j in range(nchunks - _NBUF, nchunks):
                s = j % _NBUF
                pltpu.make_async_copy(bufs[s],
                                      out_hbm.at[b, pl.ds(j * _NCH, _NCH)],
                                      osems[s]).wait()
            return carry

        lax.fori_loop(0, _BPW, batch_body, 0)

    return _k


@functools.cache
def _sc_call():
    return _make_sc_call()


def kernel(pred, y_hat):
    y0 = y_hat[..., 0]
    cc = np.arange(_W, dtype=np.float32)
    bx = jnp.asarray(np.float32(_DX) * np.tile(cc, (_H, 1)))
    by = jnp.asarray(np.float32(_DX) * np.tile(cc[:, None], (1, _W)))
    return _sc_call()(pred, y0, bx, by)


# speedup vs baseline: 21.2174x; 2.8594x over previous
import functools

import numpy as np
import jax
import jax.numpy as jnp
from jax import lax
from jax.experimental import pallas as pl
from jax.experimental.pallas import tpu as pltpu
from jax.experimental.pallas import tpu_sc as plsc

_PRIORS = (np.array([[1.3221, 1.73145], [3.19275, 4.00944], [5.05587, 8.09892],
                     [9.47112, 4.84053], [11.2364, 10.0071]],
                    dtype=np.float32) / 13.0)
_IMG = np.float32(416.0)
_B, _C, _H, _W = 64, 125, 52, 52
_HW = _H * _W
_DX = _IMG / np.float32(_C)
_NW = 32
_G = 4
_NBUF = 3
_NCHUNK = 22
_MROWS = 96
_OFFS = (0, 16, 32, 48, 64, 80, 96, 109)


def _tables():
    s = np.ones(128, np.float32)
    a = np.zeros(128, np.float32)
    bx = np.zeros(128, np.float32)
    by = np.zeros(128, np.float32)
    for c in range(_C):
        an, cm = divmod(c, 25)
        if cm == 1:
            s[c], a[c], bx[c] = _DX, 1.0, 1.0
        elif cm == 2:
            s[c], a[c], by[c] = _DX, 1.0, 1.0
        elif cm == 3:
            s[c], a[c] = _PRIORS[an, 0] * _IMG, 1.0
        elif cm == 4:
            s[c], a[c] = _PRIORS[an, 1] * _IMG, 1.0
    return s, a, bx, by


def _trunc(x):
    return x.astype(jnp.int32).astype(jnp.float32)


def _make_sc_call():
    mesh = plsc.VectorSubcoreMesh(core_axis_name="c", subcore_axis_name="s")

    @functools.partial(
        pl.kernel, mesh=mesh,
        out_type=jax.ShapeDtypeStruct((_HW, _B, _C), jnp.float32),
        scratch_types=[
            pltpu.VMEM((128,), jnp.float32),
            pltpu.VMEM((128,), jnp.float32),
            pltpu.VMEM((128,), jnp.float32),
            pltpu.VMEM((128,), jnp.float32),
            pltpu.VMEM((_MROWS, _B), jnp.float32),
            [pltpu.VMEM((_G, _B, _C), jnp.float32)] * _NBUF,
            [pltpu.SemaphoreType.DMA] * _NBUF,
            [pltpu.SemaphoreType.DMA] * _NBUF,
        ],
    )
    def _k(x_hbm, y0_hbm, s_hbm, a_hbm, bx_hbm, by_hbm, out_hbm,
           s_v, a_v, bx_v, by_v, m_all, bufs, isems, osems):
        wid = lax.axis_index("s") * 2 + lax.axis_index("c")
        start = 4 * ((676 * wid) // _NW)
        stop = 4 * ((676 * (wid + 1)) // _NW)
        hi = stop - _G
        mstart = pl.multiple_of(
            jnp.minimum(start - (start % 8), _HW - _MROWS), 8)

        def cs_of(i):
            return jnp.minimum(start + i * _G, hi)

        for j in range(_NBUF):
            pltpu.async_copy(x_hbm.at[pl.ds(cs_of(j), _G)], bufs[j], isems[j])

        pltpu.sync_copy(s_hbm, s_v)
        pltpu.sync_copy(a_hbm, a_v)
        pltpu.sync_copy(bx_hbm, bx_v)
        pltpu.sync_copy(by_hbm, by_v)
        pltpu.sync_copy(y0_hbm.at[pl.ds(mstart, _MROWS)], m_all)

        def mbody(r, c2):
            ys = [m_all[r, pl.ds(q * 16, 16)] for q in range(_B // 16)]
            for q, y in enumerate(ys):
                m_all[r, pl.ds(q * 16, 16)] = 5.0 * y + 0.5 * (1.0 - y)
            return c2
        lax.fori_loop(0, _MROWS, mbody, 0)

        sv = [s_v[pl.ds(o, 16)] for o in _OFFS]
        ab = [a_v[pl.ds(o, 16)] > 0.5 for o in _OFFS]
        bxv = [bx_v[pl.ds(o, 16)] for o in _OFFS]
        byv = [by_v[pl.ds(o, 16)] for o in _OFFS]

        def chunk_compute(buf, cs):
            def cell_body(k, c2):
                t = cs + k
                cl = t - mstart
                cyi = t // _W
                cxi = t - cyi * _W
                bxs = float(_DX) * cxi.astype(jnp.float32)
                bys = float(_DX) * cyi.astype(jnp.float32)
                bterm = [bxv[j] * bxs + byv[j] * bys for j in range(8)]
                for bg in range(_B // 16):
                    m16 = m_all[cl, pl.ds(bg * 16, 16)]

                    def b_body(bi, c3, m16=m16, bg=bg):
                        m_b = m16.at[jnp.full((16,), bi, jnp.int32)].get(
                            mode="promise_in_bounds")
                        b = bg * 16 + bi
                        xs = [buf[k, b, pl.ds(o, 16)] for o in _OFFS]
                        for j, o in enumerate(_OFFS):
                            u = _trunc(sv[j] * xs[j]) + bterm[j]
                            y = jnp.where(ab[j], u, xs[j])
                            buf[k, b, pl.ds(o, 16)] = y * m_b
                        return c3
                    lax.fori_loop(0, 16, b_body, 0)
                return c2
            lax.fori_loop(0, _G, cell_body, 0)

        for i in range(_NCHUNK):
            s = i % _NBUF
            cs = cs_of(i)
            buf = bufs[s]
            pltpu.make_async_copy(x_hbm.at[pl.ds(cs, _G)], buf,
                                  isems[s]).wait()
            chunk_compute(buf, cs)
            pltpu.async_copy(buf, out_hbm.at[pl.ds(cs, _G)], osems[s])
            ni = i + _NBUF
            if ni < _NCHUNK:
                pltpu.make_async_copy(buf, out_hbm.at[pl.ds(cs, _G)],
                                      osems[s]).wait()
                pltpu.async_copy(x_hbm.at[pl.ds(cs_of(ni), _G)], buf, isems[s])

        for i in range(_NCHUNK - _NBUF, _NCHUNK):
            s = i % _NBUF
            pltpu.make_async_copy(bufs[s], out_hbm.at[pl.ds(cs_of(i), _G)],
                                  osems[s]).wait()

    return _k


@functools.cache
def _sc_call():
    return _make_sc_call()


def kernel(pred, y_hat):
    xt = jnp.transpose(pred, (2, 3, 0, 1)).reshape(_HW, _B, _C)
    y0 = jnp.transpose(y_hat[..., 0], (1, 2, 0)).reshape(_HW, _B)
    s, a, bx, by = _tables()
    out3 = _sc_call()(xt, y0, jnp.asarray(s), jnp.asarray(a),
                      jnp.asarray(bx), jnp.asarray(by))
    return jnp.transpose(out3.reshape(_H, _W, _B, _C), (2, 3, 0, 1))


# speedup vs baseline: 21.4625x; 1.0115x over previous
import functools

import numpy as np
import jax
import jax.numpy as jnp
from jax import lax
from jax.experimental import pallas as pl
from jax.experimental.pallas import tpu as pltpu
from jax.experimental.pallas import tpu_sc as plsc

_PRIORS = (np.array([[1.3221, 1.73145], [3.19275, 4.00944], [5.05587, 8.09892],
                     [9.47112, 4.84053], [11.2364, 10.0071]],
                    dtype=np.float32) / 13.0)
_IMG = np.float32(416.0)
_B, _C, _H, _W = 64, 125, 52, 52
_HW = _H * _W
_DX = _IMG / np.float32(_C)
_NW = 32
_G = 4
_NBUF = 3
_NCHUNK = 22
_MROWS = 96
_OFFS = (0, 16, 32, 48, 64, 80, 96, 109)


def _tables():
    tab = np.zeros((4, 128), np.float32)
    tab[0] = 1.0
    for c in range(_C):
        an, cm = divmod(c, 25)
        if cm == 1:
            tab[0, c], tab[1, c], tab[2, c] = _DX, 1.0, 1.0
        elif cm == 2:
            tab[0, c], tab[1, c], tab[3, c] = _DX, 1.0, 1.0
        elif cm == 3:
            tab[0, c], tab[1, c] = _PRIORS[an, 0] * _IMG, 1.0
        elif cm == 4:
            tab[0, c], tab[1, c] = _PRIORS[an, 1] * _IMG, 1.0
    return tab


def _trunc(x):
    return x.astype(jnp.int32).astype(jnp.float32)


def _make_sc_call():
    mesh = plsc.VectorSubcoreMesh(core_axis_name="c", subcore_axis_name="s")

    @functools.partial(
        pl.kernel, mesh=mesh,
        out_type=jax.ShapeDtypeStruct((_HW, _B, _C), jnp.float32),
        scratch_types=[
            pltpu.VMEM((4, 128), jnp.float32),
            pltpu.VMEM((_MROWS, _B), jnp.float32),
            [pltpu.VMEM((_G, _B, _C), jnp.float32)] * _NBUF,
            [pltpu.SemaphoreType.DMA] * _NBUF,
            [pltpu.SemaphoreType.DMA] * _NBUF,
        ],
    )
    def _k(x_hbm, y0_hbm, tab_hbm, out_hbm, tab_v, m_all, bufs, isems, osems):
        wid = lax.axis_index("s") * 2 + lax.axis_index("c")
        start = 4 * ((676 * wid) // _NW)
        stop = 4 * ((676 * (wid + 1)) // _NW)
        hi = stop - _G
        mstart = pl.multiple_of(
            jnp.minimum(start - (start % 8), _HW - _MROWS), 8)

        def cs_of(i):
            return jnp.minimum(start + i * _G, hi)

        for j in range(_NBUF):
            pltpu.async_copy(x_hbm.at[pl.ds(cs_of(j), _G)], bufs[j], isems[j])

        pltpu.sync_copy(tab_hbm, tab_v)
        pltpu.sync_copy(y0_hbm.at[pl.ds(mstart, _MROWS)], m_all)

        def mbody(r, c2):
            ys = [m_all[r, pl.ds(q * 16, 16)] for q in range(_B // 16)]
            for q, y in enumerate(ys):
                m_all[r, pl.ds(q * 16, 16)] = 5.0 * y + 0.5 * (1.0 - y)
            return c2
        lax.fori_loop(0, _MROWS, mbody, 0)

        sv = [tab_v[0, pl.ds(o, 16)] for o in _OFFS]
        ab = [tab_v[1, pl.ds(o, 16)] > 0.5 for o in _OFFS]
        bxv = [tab_v[2, pl.ds(o, 16)] for o in _OFFS]
        byv = [tab_v[3, pl.ds(o, 16)] for o in _OFFS]

        def chunk_compute(buf, cs):
            def cell_body(k, c2):
                t = cs + k
                cl = t - mstart
                cyi = t // _W
                cxi = t - cyi * _W
                bxs = float(_DX) * cxi.astype(jnp.float32)
                bys = float(_DX) * cyi.astype(jnp.float32)
                bterm = [bxv[j] * bxs + byv[j] * bys for j in range(8)]
                for bg in range(_B // 16):
                    m16 = m_all[cl, pl.ds(bg * 16, 16)]

                    def b_body(bi, c3, m16=m16, bg=bg):
                        m_b = m16.at[jnp.full((16,), bi, jnp.int32)].get(
                            mode="promise_in_bounds")
                        b = bg * 16 + bi
                        xs = [buf[k, b, pl.ds(o, 16)] for o in _OFFS]
                        for j, o in enumerate(_OFFS):
                            u = _trunc(sv[j] * xs[j]) + bterm[j]
                            y = jnp.where(ab[j], u, xs[j])
                            buf[k, b, pl.ds(o, 16)] = y * m_b
                        return c3
                    lax.fori_loop(0, 16, b_body, 0)
                return c2
            lax.fori_loop(0, _G, cell_body, 0)

        for i in range(_NCHUNK):
            s = i % _NBUF
            cs = cs_of(i)
            buf = bufs[s]
            pltpu.make_async_copy(x_hbm.at[pl.ds(cs, _G)], buf,
                                  isems[s]).wait()
            chunk_compute(buf, cs)
            pltpu.async_copy(buf, out_hbm.at[pl.ds(cs, _G)], osems[s])
            ni = i + _NBUF
            if ni < _NCHUNK:
                pltpu.make_async_copy(buf, out_hbm.at[pl.ds(cs, _G)],
                                      osems[s]).wait()
                pltpu.async_copy(x_hbm.at[pl.ds(cs_of(ni), _G)], buf, isems[s])

        for i in range(_NCHUNK - _NBUF, _NCHUNK):
            s = i % _NBUF
            pltpu.make_async_copy(bufs[s], out_hbm.at[pl.ds(cs_of(i), _G)],
                                  osems[s]).wait()

    return _k


@functools.cache
def _sc_call():
    return _make_sc_call()


def kernel(pred, y_hat):
    xt = jnp.transpose(pred, (2, 3, 0, 1)).reshape(_HW, _B, _C)
    y0 = jnp.transpose(y_hat[..., 0], (1, 2, 0)).reshape(_HW, _B)
    out3 = _sc_call()(xt, y0, jnp.asarray(_tables()))
    return jnp.transpose(out3.reshape(_H, _W, _B, _C), (2, 3, 0, 1))


# speedup vs baseline: 25.5286x; 1.1895x over previous
import functools

import numpy as np
import jax
import jax.numpy as jnp
from jax import lax
from jax.experimental import pallas as pl
from jax.experimental.pallas import tpu as pltpu
from jax.experimental.pallas import tpu_sc as plsc

_PRIORS = (np.array([[1.3221, 1.73145], [3.19275, 4.00944], [5.05587, 8.09892],
                     [9.47112, 4.84053], [11.2364, 10.0071]],
                    dtype=np.float32) / 13.0)
_IMG = np.float32(416.0)
_B, _C, _H, _W = 64, 125, 52, 52
_HW = _H * _W
_DX = _IMG / np.float32(_C)
_NW = 32
_G = 4
_NBUF = 3
_NCHUNK = 22
_MROWS = 96
_OFFS = (0, 16, 32, 48, 64, 80, 96, 109)


def _tables():
    tab = np.zeros((4, 128), np.float32)
    tab[0] = 1.0
    for c in range(_C):
        an, cm = divmod(c, 25)
        if cm == 1:
            tab[0, c], tab[1, c], tab[2, c] = _DX, 1.0, 1.0
        elif cm == 2:
            tab[0, c], tab[1, c], tab[3, c] = _DX, 1.0, 1.0
        elif cm == 3:
            tab[0, c], tab[1, c] = _PRIORS[an, 0] * _IMG, 1.0
        elif cm == 4:
            tab[0, c], tab[1, c] = _PRIORS[an, 1] * _IMG, 1.0
    return tab


def _trunc(x):
    return x.astype(jnp.int32).astype(jnp.float32)


def _make_sc_call():
    mesh = plsc.VectorSubcoreMesh(core_axis_name="c", subcore_axis_name="s")

    @functools.partial(
        pl.kernel, mesh=mesh,
        out_type=jax.ShapeDtypeStruct((_HW, _B, _C), jnp.float32),
        scratch_types=[
            pltpu.VMEM((4, 128), jnp.float32),
            pltpu.VMEM((_MROWS, _B), jnp.float32),
            [pltpu.VMEM((_G, _B, _C), jnp.float32)] * _NBUF,
            [pltpu.SemaphoreType.DMA] * _NBUF,
            [pltpu.SemaphoreType.DMA] * _NBUF,
        ],
    )
    def _k(x_hbm, y0_hbm, tab_hbm, out_hbm, tab_v, m_all, bufs, isems, osems):
        wid = lax.axis_index("s") * 2 + lax.axis_index("c")
        start = 4 * ((676 * wid) // _NW)
        stop = 4 * ((676 * (wid + 1)) // _NW)
        hi = stop - _G
        mstart = pl.multiple_of(
            jnp.minimum(start - (start % 8), _HW - _MROWS), 8)

        def cs_of(i):
            return jnp.minimum(start + i * _G, hi)

        for j in range(_NBUF):
            pltpu.async_copy(x_hbm.at[pl.ds(cs_of(j), _G)], bufs[j], isems[j])

        pltpu.sync_copy(tab_hbm, tab_v)
        pltpu.sync_copy(y0_hbm.at[pl.ds(mstart, _MROWS)], m_all)

        def mbody(r, c2):
            ys = [m_all[r, pl.ds(q * 16, 16)] for q in range(_B // 16)]
            for q, y in enumerate(ys):
                m_all[r, pl.ds(q * 16, 16)] = 5.0 * y + 0.5 * (1.0 - y)
            return c2
        lax.fori_loop(0, _MROWS, mbody, 0)

        sv = [tab_v[0, pl.ds(o, 16)] for o in _OFFS]
        ab = [tab_v[1, pl.ds(o, 16)] > 0.5 for o in _OFFS]
        bxv = [tab_v[2, pl.ds(o, 16)] for o in _OFFS]
        byv = [tab_v[3, pl.ds(o, 16)] for o in _OFFS]

        def chunk_compute(buf, cs):
            def cell_body(k, c2):
                t = cs + k
                cl = t - mstart
                cyi = t // _W
                cxi = t - cyi * _W
                bxs = float(_DX) * cxi.astype(jnp.float32)
                bys = float(_DX) * cyi.astype(jnp.float32)
                bterm = [bxv[j] * bxs + byv[j] * bys for j in range(8)]
                for bg in range(_B // 16):
                    m16 = m_all[cl, pl.ds(bg * 16, 16)]

                    def b_body(bi, c3, m16=m16, bg=bg):
                        m_b = m16.at[jnp.full((16,), bi, jnp.int32)].get(
                            mode="promise_in_bounds")
                        b = bg * 16 + bi
                        xs = [buf[k, b, pl.ds(o, 16)] for o in _OFFS]
                        for j, o in enumerate(_OFFS):
                            u = _trunc(sv[j] * xs[j]) + bterm[j]
                            y = jnp.where(ab[j], u, xs[j])
                            buf[k, b, pl.ds(o, 16)] = y * m_b
                        return c3
                    lax.fori_loop(0, 16, b_body, 0)
                return c2
            lax.fori_loop(0, _G, cell_body, 0)

        for i in range(_NCHUNK):
            s = i % _NBUF
            cs = cs_of(i)
            buf = bufs[s]
            pltpu.make_async_copy(x_hbm.at[pl.ds(cs, _G)], buf,
                                  isems[s]).wait()
            chunk_compute(buf, cs)
            pltpu.async_copy(buf, out_hbm.at[pl.ds(cs, _G)], osems[s])
            ni = i - 1 + _NBUF
            if i >= 1 and ni < _NCHUNK:
                sp = (i - 1) % _NBUF
                pltpu.make_async_copy(bufs[sp],
                                      out_hbm.at[pl.ds(cs_of(i - 1), _G)],
                                      osems[sp]).wait()
                pltpu.async_copy(x_hbm.at[pl.ds(cs_of(ni), _G)], bufs[sp],
                                 isems[sp])

        for i in range(_NCHUNK - _NBUF, _NCHUNK):
            s = i % _NBUF
            pltpu.make_async_copy(bufs[s], out_hbm.at[pl.ds(cs_of(i), _G)],
                                  osems[s]).wait()

    return _k


@functools.cache
def _sc_call():
    return _make_sc_call()


def kernel(pred, y_hat):
    xt = jnp.transpose(pred, (2, 3, 0, 1)).reshape(_HW, _B, _C)
    y0 = jnp.transpose(y_hat[..., 0], (1, 2, 0)).reshape(_HW, _B)
    out3 = _sc_call()(xt, y0, jnp.asarray(_tables()))
    return jnp.transpose(out3.reshape(_H, _W, _B, _C), (2, 3, 0, 1))
